# baseline (speedup 1.0000x reference)
import functools

import jax
import jax.numpy as jnp
from jax import lax
from jax.experimental import pallas as pl
from jax.experimental.pallas import tpu as pltpu
from jax.experimental.pallas import tpu_sc as plsc

B = 16384
NF = 48
NG = 6
MF = 16
NC = 2
NS = 16
NW = NC * NS
RPW = B // NW
PK = 8
RING = 4
AHEAD = 2
NSLOT = RPW // PK

V = 1000000


_sc_mesh = plsc.VectorSubcoreMesh(core_axis_name="c", subcore_axis_name="s")


@functools.partial(
    pl.kernel,
    mesh=_sc_mesh,
    out_type=[
        jax.ShapeDtypeStruct((NF, B), jnp.float32),
        jax.ShapeDtypeStruct((NF, B), jnp.float32),
    ],
    scratch_types=[
        pltpu.SMEM((RPW,), jnp.int32),
        pltpu.SMEM((RPW,), jnp.int32),
        pltpu.VMEM((RPW,), jnp.int32),
        pltpu.VMEM((RPW,), jnp.int32),
        pltpu.VMEM_SHARED((NS, NC, 2 * RPW), jnp.int32),
        pltpu.VMEM((RING, NG, 8, 128), jnp.float32),
        pltpu.VMEM((RING, NG, 8, 128), jnp.float32),
        pltpu.VMEM((NF, RPW), jnp.float32),
        pltpu.VMEM((NF, RPW), jnp.float32),
        pltpu.VMEM((RPW,), jnp.int32),
        pltpu.VMEM((RPW,), jnp.int32),
        pltpu.SemaphoreType.DMA,
        pltpu.SemaphoreType.DMA,
    ],
    compiler_params=pltpu.CompilerParams(
        use_tc_tiling_on_sc=True, needs_layout_passes=False),
)
def _sc_gather(ut3, uidx_hbm, it3, iidx_hbm, uout, iout,
               uidx_s, iidx_s, uidx_v, iidx_v, idx_sh,
               ustage, istage, urows, irows, ulane, ilane, usem, isem):
    cid = lax.axis_index("c")
    sid = lax.axis_index("s")
    wid = sid * NC + cid
    base = wid * RPW

    pltpu.sync_copy(uidx_hbm.at[pl.ds(base, RPW)], uidx_v)
    pltpu.sync_copy(iidx_hbm.at[pl.ds(base, RPW)], iidx_v)
    pltpu.sync_copy(uidx_v, idx_sh.at[sid, cid, pl.ds(0, RPW)])
    pltpu.sync_copy(iidx_v, idx_sh.at[sid, cid, pl.ds(RPW, RPW)])
    pltpu.sync_copy(idx_sh.at[sid, cid, pl.ds(0, RPW)], uidx_s)
    pltpu.sync_copy(idx_sh.at[sid, cid, pl.ds(RPW, RPW)], iidx_s)

    lanes = lax.iota(jnp.int32, 16)
    goff = lanes // 8
    f8off = lanes % 8

    jpat = (lanes % PK) * 16
    for c in range(RPW // 16):
        sl = pl.ds(c * 16, 16)
        ulane[sl] = (uidx_v[sl] & 15) + jpat
        ilane[sl] = (iidx_v[sl] & 15) + jpat

    def fire(s):
        slot = s % RING
        for j in range(PK):
            r = s * PK + j
            ub = pl.multiple_of((uidx_s[r] // 16) * 16, 16)
            ib = pl.multiple_of((iidx_s[r] // 16) * 16, 16)
            pltpu.async_copy(
                ut3.at[:, :, pl.ds(ub, 16)],
                ustage.at[slot, :, :, pl.ds(j * 16, 16)], usem)
            pltpu.async_copy(
                it3.at[:, :, pl.ds(ib, 16)],
                istage.at[slot, :, :, pl.ds(j * 16, 16)], isem)

    def drain(s):
        slot = s % RING
        pltpu.make_async_copy(
            ut3.at[:, :, pl.ds(0, 128)], ustage.at[slot], usem).wait()
        pltpu.make_async_copy(
            it3.at[:, :, pl.ds(0, 128)], istage.at[slot], isem).wait()

    feat0 = goff * 8 + f8off

    def extract(s):
        slot = s % RING
        slot_v = jnp.full((16,), slot, jnp.int32)
        s_v = jnp.full((16,), s, jnp.int32)
        loads = []
        for j in range(PK):
            r_v = s_v * PK + j
            for lane_arr, stage, rows in ((ulane, ustage, urows),
                                          (ilane, istage, irows)):
                lane_v = plsc.load_gather(lane_arr, [r_v])
                for t in range(3):
                    vals = plsc.load_gather(
                        stage, [slot_v, goff + 2 * t, f8off, lane_v])
                    loads.append((rows, feat0 + 16 * t, r_v, vals))
        for rows, feat_v, col_v, vals in loads:
            plsc.store_scatter(rows, [feat_v, col_v], vals)

    for p in range(AHEAD):
        fire(p)

    def body(s, carry):
        @pl.when(s < NSLOT - AHEAD)
        def _():
            fire(s + AHEAD)

        drain(s)
        extract(s)
        return carry

    lax.fori_loop(0, NSLOT, body, 0)

    pltpu.sync_copy(urows, uout.at[:, pl.ds(base, RPW)])
    pltpu.sync_copy(irows, iout.at[:, pl.ds(base, RPW)])


def _mlp_body(u_ref, i_ref, w1_ref, b1_ref, w2_ref, b2_ref, w3_ref, b3_ref,
              wp_ref, bp_ref, out_ref):
    u = u_ref[...]
    it = i_ref[...]
    mf = u[:MF, :] * it[:MF, :]
    x = jnp.concatenate([u[MF:, :], it[MF:, :]], axis=0)
    dn = (((0,), (0,)), ((), ()))
    x = jnp.maximum(
        lax.dot_general(w1_ref[...], x, dn,
                        preferred_element_type=jnp.float32) + b1_ref[...], 0.0)
    x = jnp.maximum(
        lax.dot_general(w2_ref[...], x, dn,
                        preferred_element_type=jnp.float32) + b2_ref[...], 0.0)
    x = jnp.maximum(
        lax.dot_general(w3_ref[...], x, dn,
                        preferred_element_type=jnp.float32) + b3_ref[...], 0.0)
    pred = jnp.concatenate([mf, x], axis=0)
    logits = lax.dot_general(wp_ref[...], pred, dn,
                             preferred_element_type=jnp.float32) + bp_ref[...]
    out_ref[...] = jnp.concatenate([jnp.zeros_like(logits), logits], axis=0)


NBLK = 8
BLK = B // NBLK


_mlp = pl.pallas_call(
    _mlp_body,
    grid=(NBLK,),
    in_specs=[
        pl.BlockSpec((NF, BLK), lambda i: (0, i)),
        pl.BlockSpec((NF, BLK), lambda i: (0, i)),
        pl.BlockSpec((64, 32), lambda i: (0, 0)),
        pl.BlockSpec((32, 1), lambda i: (0, 0)),
        pl.BlockSpec((32, 16), lambda i: (0, 0)),
        pl.BlockSpec((16, 1), lambda i: (0, 0)),
        pl.BlockSpec((16, 8), lambda i: (0, 0)),
        pl.BlockSpec((8, 1), lambda i: (0, 0)),
        pl.BlockSpec((24, 1), lambda i: (0, 0)),
        pl.BlockSpec((1, 1), lambda i: (0, 0)),
    ],
    out_specs=pl.BlockSpec((2, BLK), lambda i: (0, i)),
    out_shape=jax.ShapeDtypeStruct((2, B), jnp.float32),
)


def kernel(user_input, item_input, user_table, item_table,
           W1, b1, W2, b2, W3, b3, Wp, bp):
    uidx = user_input[:, 0]
    iidx = item_input[:, 0]
    ut3 = user_table.T.reshape(NG, 8, V)
    it3 = item_table.T.reshape(NG, 8, V)
    ug, ig = _sc_gather(ut3, uidx, it3, iidx)
    out_t = _mlp(ug, ig,
                 W1, b1.reshape(32, 1),
                 W2, b2.reshape(16, 1),
                 W3, b3.reshape(8, 1),
                 Wp, bp.reshape(1, 1))
    return out_t.T


# speedup vs baseline: 1.0267x; 1.0267x over previous
import functools

import jax
import jax.numpy as jnp
from jax import lax
from jax.experimental import pallas as pl
from jax.experimental.pallas import tpu as pltpu
from jax.experimental.pallas import tpu_sc as plsc

B = 16384
NF = 48
NG = 6
MF = 16
NC = 2
NS = 16
NW = NC * NS
RPW = B // NW
PK = 8
RING = 4
AHEAD = 2
NSLOT = RPW // PK

V = 1000000


_sc_mesh = plsc.VectorSubcoreMesh(core_axis_name="c", subcore_axis_name="s")


@functools.partial(
    pl.kernel,
    mesh=_sc_mesh,
    out_type=[
        jax.ShapeDtypeStruct((NF, B), jnp.float32),
        jax.ShapeDtypeStruct((NF, B), jnp.float32),
    ],
    scratch_types=[
        pltpu.SMEM((RPW,), jnp.int32),
        pltpu.SMEM((RPW,), jnp.int32),
        pltpu.VMEM((RPW,), jnp.int32),
        pltpu.VMEM((RPW,), jnp.int32),
        pltpu.VMEM_SHARED((NS, NC, 2 * RPW), jnp.int32),
        pltpu.VMEM((RING, NG, 8, 128), jnp.float32),
        pltpu.VMEM((RING, NG, 8, 128), jnp.float32),
        pltpu.VMEM((NF, RPW), jnp.float32),
        pltpu.VMEM((NF, RPW), jnp.float32),
        pltpu.VMEM((RPW,), jnp.int32),
        pltpu.VMEM((RPW,), jnp.int32),
        pltpu.SemaphoreType.DMA,
        pltpu.SemaphoreType.DMA,
    ],
    compiler_params=pltpu.CompilerParams(
        use_tc_tiling_on_sc=True, needs_layout_passes=False),
)
def _sc_gather(ut3, uidx_hbm, it3, iidx_hbm, uout, iout,
               uidx_s, iidx_s, uidx_v, iidx_v, idx_sh,
               ustage, istage, urows, irows, ulane, ilane, usem, isem):
    cid = lax.axis_index("c")
    sid = lax.axis_index("s")
    wid = sid * NC + cid
    base = wid * RPW

    pltpu.sync_copy(uidx_hbm.at[pl.ds(base, RPW)], uidx_v)
    pltpu.sync_copy(iidx_hbm.at[pl.ds(base, RPW)], iidx_v)
    pltpu.sync_copy(uidx_v, idx_sh.at[sid, cid, pl.ds(0, RPW)])
    pltpu.sync_copy(iidx_v, idx_sh.at[sid, cid, pl.ds(RPW, RPW)])
    pltpu.sync_copy(idx_sh.at[sid, cid, pl.ds(0, RPW)], uidx_s)
    pltpu.sync_copy(idx_sh.at[sid, cid, pl.ds(RPW, RPW)], iidx_s)

    lanes = lax.iota(jnp.int32, 16)
    goff = lanes // 8
    f8off = lanes % 8

    jpat = (lanes % PK) * 16
    for c in range(RPW // 16):
        sl = pl.ds(c * 16, 16)
        ulane[sl] = (uidx_v[sl] & 15) + jpat
        ilane[sl] = (iidx_v[sl] & 15) + jpat

    def fire(s):
        slot = s % RING
        for j in range(PK):
            r = s * PK + j
            ub = pl.multiple_of((uidx_s[r] // 16) * 16, 16)
            ib = pl.multiple_of((iidx_s[r] // 16) * 16, 16)
            pltpu.async_copy(
                ut3.at[:, :, pl.ds(ub, 16)],
                ustage.at[slot, :, :, pl.ds(j * 16, 16)], usem)
            pltpu.async_copy(
                it3.at[:, :, pl.ds(ib, 16)],
                istage.at[slot, :, :, pl.ds(j * 16, 16)], isem)

    def drain(s):
        slot = s % RING
        pltpu.make_async_copy(
            ut3.at[:, :, pl.ds(0, 128)], ustage.at[slot], usem).wait()
        pltpu.make_async_copy(
            it3.at[:, :, pl.ds(0, 128)], istage.at[slot], isem).wait()

    feat0 = goff * 8 + f8off

    def extract(s):
        slot = s % RING
        slot_v = jnp.full((16,), slot, jnp.int32)
        s_v = jnp.full((16,), s, jnp.int32)
        loads = []
        for j in range(PK):
            r_v = s_v * PK + j
            for lane_arr, stage, rows in ((ulane, ustage, urows),
                                          (ilane, istage, irows)):
                lane_v = plsc.load_gather(lane_arr, [r_v])
                for t in range(3):
                    vals = plsc.load_gather(
                        stage, [slot_v, goff + 2 * t, f8off, lane_v])
                    loads.append((rows, feat0 + 16 * t, r_v, vals))
        for rows, feat_v, col_v, vals in loads:
            plsc.store_scatter(rows, [feat_v, col_v], vals)

    for p in range(AHEAD):
        fire(p)

    def body(s, carry):
        @pl.when(s < NSLOT - AHEAD)
        def _():
            fire(s + AHEAD)

        drain(s)
        extract(s)
        return carry

    lax.fori_loop(0, NSLOT, body, 0)

    pltpu.sync_copy(urows, uout.at[:, pl.ds(base, RPW)])
    pltpu.sync_copy(irows, iout.at[:, pl.ds(base, RPW)])


def _mlp_body(u_ref, i_ref, w1_ref, b1_ref, w2_ref, b2_ref, w3_ref, b3_ref,
              wp_ref, bp_ref, out_ref):
    u = u_ref[...]
    it = i_ref[...]
    mf = u[:MF, :] * it[:MF, :]
    x = jnp.concatenate([u[MF:, :], it[MF:, :]], axis=0)
    dn = (((0,), (0,)), ((), ()))
    x = jnp.maximum(
        lax.dot_general(w1_ref[...], x, dn,
                        preferred_element_type=jnp.float32) + b1_ref[...], 0.0)
    x = jnp.maximum(
        lax.dot_general(w2_ref[...], x, dn,
                        preferred_element_type=jnp.float32) + b2_ref[...], 0.0)
    x = jnp.maximum(
        lax.dot_general(w3_ref[...], x, dn,
                        preferred_element_type=jnp.float32) + b3_ref[...], 0.0)
    pred = jnp.concatenate([mf, x], axis=0)
    logits = lax.dot_general(wp_ref[...], pred, dn,
                             preferred_element_type=jnp.float32) + bp_ref[...]
    out_ref[...] = jnp.concatenate([jnp.zeros_like(logits), logits], axis=0)


NBLK = 2
BLK = B // NBLK


_mlp = pl.pallas_call(
    _mlp_body,
    grid=(NBLK,),
    in_specs=[
        pl.BlockSpec((NF, BLK), lambda i: (0, i)),
        pl.BlockSpec((NF, BLK), lambda i: (0, i)),
        pl.BlockSpec((64, 32), lambda i: (0, 0)),
        pl.BlockSpec((32, 1), lambda i: (0, 0)),
        pl.BlockSpec((32, 16), lambda i: (0, 0)),
        pl.BlockSpec((16, 1), lambda i: (0, 0)),
        pl.BlockSpec((16, 8), lambda i: (0, 0)),
        pl.BlockSpec((8, 1), lambda i: (0, 0)),
        pl.BlockSpec((24, 1), lambda i: (0, 0)),
        pl.BlockSpec((1, 1), lambda i: (0, 0)),
    ],
    out_specs=pl.BlockSpec((2, BLK), lambda i: (0, i)),
    out_shape=jax.ShapeDtypeStruct((2, B), jnp.float32),
)


def kernel(user_input, item_input, user_table, item_table,
           W1, b1, W2, b2, W3, b3, Wp, bp):
    uidx = user_input[:, 0]
    iidx = item_input[:, 0]
    ut3 = user_table.T.reshape(NG, 8, V)
    it3 = item_table.T.reshape(NG, 8, V)
    ug, ig = _sc_gather(ut3, uidx, it3, iidx)
    out_t = _mlp(ug, ig,
                 W1, b1.reshape(32, 1),
                 W2, b2.reshape(16, 1),
                 W3, b3.reshape(8, 1),
                 Wp, bp.reshape(1, 1))
    return out_t.T
